# baseline (speedup 1.0000x reference)
import jax
import jax.numpy as jnp
from jax import lax
from jax.experimental import pallas as pl
from jax.experimental.pallas import tpu as pltpu
from jax.experimental.pallas import tpu_sc as plsc

D = 128
B = 4096
S = 200
NUM_SEG = 2
N = B * S

NUM_CORES = 2
NUM_SUBCORES = 16
NUM_WORKERS = NUM_CORES * NUM_SUBCORES
PER_W = N // NUM_WORKERS
CHUNK = 128
CHUNKS_PER_W = PER_W // CHUNK
LANES = 16
VPR = D // LANES


def _body(ids_hbm, seg_hbm, w_hbm, p_hbm, sg_hbm, out_hbm,
          ps_shared, pbuf_v, sg_v,
          ids0, ids1, segs0, segs1, psidx0, psidx1,
          rows0, rows1, psrows0, psrows1,
          gsem0, gsem1, pssem0, pssem1, isem0, isem1,
          ssem0, ssem1, wsem0, wsem1):
    ids_v = (ids0, ids1)
    segs_v = (segs0, segs1)
    psidx_v = (psidx0, psidx1)
    rows_v = (rows0, rows1)
    psrows_v = (psrows0, psrows1)
    gsem = (gsem0, gsem1)
    pssem = (pssem0, pssem1)
    isem = (isem0, isem1)
    ssem = (ssem0, ssem1)
    wsem = (wsem0, wsem1)

    cid = lax.axis_index("c")
    sid = lax.axis_index("s")
    wid = sid * NUM_CORES + cid
    wstart = wid * PER_W

    def chunk_base(n):
        return lax.rem(wstart + n * CHUNK, N)

    def fetch_ids(n, p):
        base = chunk_base(n)
        pltpu.async_copy(ids_hbm.at[pl.ds(base, CHUNK)], ids_v[p], isem[p])
        pltpu.async_copy(seg_hbm.at[pl.ds(base, CHUNK)], segs_v[p], ssem[p])

    def wait_ids(n, p):
        base = chunk_base(n)
        pltpu.make_async_copy(ids_hbm.at[pl.ds(base, CHUNK)], ids_v[p],
                              isem[p]).wait()
        pltpu.make_async_copy(seg_hbm.at[pl.ds(base, CHUNK)], segs_v[p],
                              ssem[p]).wait()

    def issue_gathers(n, p):
        base = chunk_base(n)
        for k in range(CHUNK // LANES):
            sl = pl.ds(LANES * k, LANES)
            svec = lax.rem(jnp.full((LANES,), base + LANES * k, jnp.int32)
                           + lax.iota(jnp.int32, LANES), S)
            psidx_v[p][sl] = segs_v[p][sl] * S + svec
        pltpu.async_copy(w_hbm.at[ids_v[p]], rows_v[p], gsem[p])
        pltpu.async_copy(ps_shared.at[psidx_v[p]], psrows_v[p], pssem[p])

    def wait_gathers(p):
        pltpu.make_async_copy(w_hbm.at[ids_v[p]], rows_v[p], gsem[p]).wait()
        pltpu.make_async_copy(ps_shared.at[psidx_v[p]], psrows_v[p],
                              pssem[p]).wait()

    def add_and_writeout(n, p):
        rv, pv = rows_v[p], psrows_v[p]
        base = chunk_base(n)
        half = CHUNK // 2

        @pl.loop(0, CHUNK, unroll=2)
        def _row(r):
            for j in range(VPR):
                sl = pl.ds(16 * j, 16)
                rv[r, sl] = rv[r, sl] + pv[r, sl]

        pltpu.async_copy(rv, out_hbm.at[pl.ds(base, CHUNK)], wsem[p])

    def wait_writeout(n, p):
        pltpu.make_async_copy(rows_v[p],
                              out_hbm.at[pl.ds(chunk_base(n), CHUNK)],
                              wsem[p]).wait()

    fetch_ids(0, 0)
    fetch_ids(1, 1)

    @pl.when(sid == 0)
    def _build():
        pltpu.sync_copy(p_hbm.at[pl.ds(0, S)], pbuf_v)
        pltpu.sync_copy(sg_hbm, sg_v)
        seg0 = [sg_v[0, pl.ds(16 * j, 16)] for j in range(VPR)]
        dseg = [sg_v[1, pl.ds(16 * j, 16)] - sg_v[0, pl.ds(16 * j, 16)]
                for j in range(VPR)]

        @pl.loop(0, S)
        def _add0(s):
            for j in range(VPR):
                sl = pl.ds(16 * j, 16)
                pbuf_v[s, sl] = pbuf_v[s, sl] + seg0[j]

        pltpu.sync_copy(pbuf_v, ps_shared.at[pl.ds(0, S)])

        @pl.loop(0, S)
        def _add1(s):
            for j in range(VPR):
                sl = pl.ds(16 * j, 16)
                pbuf_v[s, sl] = pbuf_v[s, sl] + dseg[j]

        pltpu.sync_copy(pbuf_v, ps_shared.at[pl.ds(S, S)])

    plsc.subcore_barrier()

    wait_ids(0, 0)
    issue_gathers(0, 0)
    wait_ids(1, 1)
    issue_gathers(1, 1)
    wait_gathers(0)
    fetch_ids(2, 0)
    add_and_writeout(0, 0)

    @pl.loop(1, CHUNKS_PER_W // 2)
    def _pair(m):
        for b in range(2):
            p, q = b, 1 - b
            n = 2 * m + b
            wait_writeout(n - 2, p)
            wait_ids(n, p)
            issue_gathers(n, p)
            wait_gathers(q)
            fetch_ids(n + 1, q)
            add_and_writeout(n - 1, q)

    wait_writeout(CHUNKS_PER_W - 2, 0)
    wait_gathers(1)
    add_and_writeout(CHUNKS_PER_W - 1, 1)
    wait_ids(CHUNKS_PER_W, 0)
    wait_writeout(CHUNKS_PER_W - 1, 1)


@jax.jit
def _run(input_ids, segment_ids, word_embeddings, position_embeddings,
         segment_embeddings):
    ids = input_ids.reshape(N)
    segs = segment_ids.reshape(N)
    mesh = plsc.VectorSubcoreMesh(core_axis_name="c", subcore_axis_name="s",
                                  num_cores=NUM_CORES,
                                  num_subcores=NUM_SUBCORES)
    out = pl.kernel(
        _body,
        out_type=jax.ShapeDtypeStruct((N, D), jnp.float32),
        mesh=mesh,
        scratch_types=[
            pltpu.VMEM_SHARED((NUM_SEG * S, D), jnp.float32),
            pltpu.VMEM((S, D), jnp.float32),
            pltpu.VMEM((NUM_SEG, D), jnp.float32),
            pltpu.VMEM((CHUNK,), jnp.int32),
            pltpu.VMEM((CHUNK,), jnp.int32),
            pltpu.VMEM((CHUNK,), jnp.int32),
            pltpu.VMEM((CHUNK,), jnp.int32),
            pltpu.VMEM((CHUNK,), jnp.int32),
            pltpu.VMEM((CHUNK,), jnp.int32),
            pltpu.VMEM((CHUNK, D), jnp.float32),
            pltpu.VMEM((CHUNK, D), jnp.float32),
            pltpu.VMEM((CHUNK, D), jnp.float32),
            pltpu.VMEM((CHUNK, D), jnp.float32),
        ] + [pltpu.SemaphoreType.DMA] * 10,
    )(ids, segs, word_embeddings, position_embeddings, segment_embeddings)
    return out.reshape(B, S, D)


def kernel(input_ids, segment_ids, word_embeddings, position_embeddings,
           segment_embeddings):
    return _run(input_ids, segment_ids, word_embeddings,
                position_embeddings, segment_embeddings)


# speedup vs baseline: 2.6850x; 2.6850x over previous
import jax
import jax.numpy as jnp
from jax import lax
from jax.experimental import pallas as pl
from jax.experimental.pallas import tpu as pltpu
from jax.experimental.pallas import tpu_sc as plsc

D = 128
B = 4096
S = 200
NUM_SEG = 2
N = B * S

NUM_CORES = 2
NUM_SUBCORES = 16
NUM_WORKERS = NUM_CORES * NUM_SUBCORES
PER_W = N // NUM_WORKERS
CHUNK = 128
CHUNKS_PER_W = PER_W // CHUNK
LANES = 16
VPR = D // LANES


def _body(ids_hbm, seg_hbm, w_hbm, p_hbm, sg_hbm, out_hbm,
          ps_shared, pbuf_v, sg_v,
          ids0, ids1, segs0, segs1, psidx0, psidx1,
          rows0, rows1, psrows0, psrows1,
          gsem0, gsem1, pssem0, pssem1, isem0, isem1,
          ssem0, ssem1, wsem0, wsem1):
    ids_v = (ids0, ids1)
    segs_v = (segs0, segs1)
    psidx_v = (psidx0, psidx1)
    rows_v = (rows0, rows1)
    psrows_v = (psrows0, psrows1)
    gsem = (gsem0, gsem1)
    pssem = (pssem0, pssem1)
    isem = (isem0, isem1)
    ssem = (ssem0, ssem1)
    wsem = (wsem0, wsem1)

    cid = lax.axis_index("c")
    sid = lax.axis_index("s")
    wid = sid * NUM_CORES + cid
    wstart = wid * PER_W

    def chunk_base(n):
        return lax.rem(wstart + n * CHUNK, N)

    def fetch_ids(n, p):
        base = chunk_base(n)
        pltpu.async_copy(ids_hbm.at[pl.ds(base, CHUNK)], ids_v[p], isem[p])
        pltpu.async_copy(seg_hbm.at[pl.ds(base, CHUNK)], segs_v[p], ssem[p])

    def wait_ids(n, p):
        base = chunk_base(n)
        pltpu.make_async_copy(ids_hbm.at[pl.ds(base, CHUNK)], ids_v[p],
                              isem[p]).wait()
        pltpu.make_async_copy(seg_hbm.at[pl.ds(base, CHUNK)], segs_v[p],
                              ssem[p]).wait()

    def issue_gathers(n, p):
        base = chunk_base(n)
        for k in range(CHUNK // LANES):
            sl = pl.ds(LANES * k, LANES)
            svec = lax.rem(jnp.full((LANES,), base + LANES * k, jnp.int32)
                           + lax.iota(jnp.int32, LANES), S)
            psidx_v[p][sl] = segs_v[p][sl] * S + svec
        pltpu.async_copy(w_hbm.at[ids_v[p]], rows_v[p], gsem[p])
        pltpu.async_copy(ps_shared.at[psidx_v[p]], psrows_v[p], pssem[p])

    def wait_gathers(p):
        pltpu.make_async_copy(w_hbm.at[ids_v[p]], rows_v[p], gsem[p]).wait()
        pltpu.make_async_copy(ps_shared.at[psidx_v[p]], psrows_v[p],
                              pssem[p]).wait()

    def add_and_writeout(n, p):
        rv, pv = rows_v[p], psrows_v[p]
        base = chunk_base(n)
        half = CHUNK // 2

        @pl.loop(0, CHUNK)
        def _row(r):
            for j in range(VPR):
                sl = pl.ds(16 * j, 16)
                rv[r, sl] = rv[r, sl] + pv[r, sl]

        pltpu.async_copy(rv, out_hbm.at[pl.ds(base, CHUNK)], wsem[p])

    def wait_writeout(n, p):
        pltpu.make_async_copy(rows_v[p],
                              out_hbm.at[pl.ds(chunk_base(n), CHUNK)],
                              wsem[p]).wait()

    fetch_ids(0, 0)
    fetch_ids(1, 1)

    @pl.when(sid == 0)
    def _build():
        pltpu.sync_copy(p_hbm.at[pl.ds(0, S)], pbuf_v)
        pltpu.sync_copy(sg_hbm, sg_v)
        seg0 = [sg_v[0, pl.ds(16 * j, 16)] for j in range(VPR)]
        dseg = [sg_v[1, pl.ds(16 * j, 16)] - sg_v[0, pl.ds(16 * j, 16)]
                for j in range(VPR)]

        @pl.loop(0, S)
        def _add0(s):
            for j in range(VPR):
                sl = pl.ds(16 * j, 16)
                pbuf_v[s, sl] = pbuf_v[s, sl] + seg0[j]

        pltpu.sync_copy(pbuf_v, ps_shared.at[pl.ds(0, S)])

        @pl.loop(0, S)
        def _add1(s):
            for j in range(VPR):
                sl = pl.ds(16 * j, 16)
                pbuf_v[s, sl] = pbuf_v[s, sl] + dseg[j]

        pltpu.sync_copy(pbuf_v, ps_shared.at[pl.ds(S, S)])

    plsc.subcore_barrier()

    wait_ids(0, 0)
    issue_gathers(0, 0)
    wait_ids(1, 1)
    issue_gathers(1, 1)
    wait_gathers(0)
    fetch_ids(2, 0)
    add_and_writeout(0, 0)

    @pl.loop(1, CHUNKS_PER_W // 2)
    def _pair(m):
        for b in range(2):
            p, q = b, 1 - b
            n = 2 * m + b
            wait_writeout(n - 2, p)
            wait_ids(n, p)
            issue_gathers(n, p)
            wait_gathers(q)
            fetch_ids(n + 1, q)
            add_and_writeout(n - 1, q)

    wait_writeout(CHUNKS_PER_W - 2, 0)
    wait_gathers(1)
    add_and_writeout(CHUNKS_PER_W - 1, 1)
    wait_ids(CHUNKS_PER_W, 0)
    wait_writeout(CHUNKS_PER_W - 1, 1)


@jax.jit
def _run(input_ids, segment_ids, word_embeddings, position_embeddings,
         segment_embeddings):
    ids = input_ids.reshape(N)
    segs = segment_ids.reshape(N)
    mesh = plsc.VectorSubcoreMesh(core_axis_name="c", subcore_axis_name="s",
                                  num_cores=NUM_CORES,
                                  num_subcores=NUM_SUBCORES)
    out = pl.kernel(
        _body,
        out_type=jax.ShapeDtypeStruct((N, D), jnp.float32),
        mesh=mesh,
        scratch_types=[
            pltpu.VMEM_SHARED((NUM_SEG * S, D), jnp.float32),
            pltpu.VMEM((S, D), jnp.float32),
            pltpu.VMEM((NUM_SEG, D), jnp.float32),
            pltpu.VMEM((CHUNK,), jnp.int32),
            pltpu.VMEM((CHUNK,), jnp.int32),
            pltpu.VMEM((CHUNK,), jnp.int32),
            pltpu.VMEM((CHUNK,), jnp.int32),
            pltpu.VMEM((CHUNK,), jnp.int32),
            pltpu.VMEM((CHUNK,), jnp.int32),
            pltpu.VMEM((CHUNK, D), jnp.float32),
            pltpu.VMEM((CHUNK, D), jnp.float32),
            pltpu.VMEM((CHUNK, D), jnp.float32),
            pltpu.VMEM((CHUNK, D), jnp.float32),
        ] + [pltpu.SemaphoreType.DMA] * 10,
    )(ids, segs, word_embeddings, position_embeddings, segment_embeddings)
    return out.reshape(B, S, D)


def kernel(input_ids, segment_ids, word_embeddings, position_embeddings,
           segment_embeddings):
    return _run(input_ids, segment_ids, word_embeddings,
                position_embeddings, segment_embeddings)


# speedup vs baseline: 3.0823x; 1.1480x over previous
import jax
import jax.numpy as jnp
from jax import lax
from jax.experimental import pallas as pl
from jax.experimental.pallas import tpu as pltpu
from jax.experimental.pallas import tpu_sc as plsc

D = 128
B = 4096
S = 200
NUM_SEG = 2
N = B * S

NUM_CORES = 2
NUM_SUBCORES = 16
NUM_WORKERS = NUM_CORES * NUM_SUBCORES
PER_W = N // NUM_WORKERS
CHUNK = 128
CHUNKS_PER_W = PER_W // CHUNK
LANES = 16
VPR = D // LANES
DEPTH = 3


def _body(ids_hbm, seg_hbm, w_hbm, p_hbm, sg_hbm, out_hbm,
          ps_shared, pbuf_v, sg_v,
          ids0, ids1, ids2, segs0, segs1, segs2, psidx0, psidx1, psidx2,
          rows0, rows1, rows2, psrows0, psrows1, psrows2,
          gsem0, gsem1, gsem2, pssem0, pssem1, pssem2,
          isem0, isem1, isem2, ssem0, ssem1, ssem2,
          wsem0, wsem1, wsem2):
    ids_v = (ids0, ids1, ids2)
    segs_v = (segs0, segs1, segs2)
    psidx_v = (psidx0, psidx1, psidx2)
    rows_v = (rows0, rows1, rows2)
    psrows_v = (psrows0, psrows1, psrows2)
    gsem = (gsem0, gsem1, gsem2)
    pssem = (pssem0, pssem1, pssem2)
    isem = (isem0, isem1, isem2)
    ssem = (ssem0, ssem1, ssem2)
    wsem = (wsem0, wsem1, wsem2)

    cid = lax.axis_index("c")
    sid = lax.axis_index("s")
    wid = sid * NUM_CORES + cid
    wstart = wid * PER_W

    def chunk_base(n):
        return lax.rem(wstart + n * CHUNK, N)

    def fetch_ids(n, p):
        base = chunk_base(n)
        pltpu.async_copy(ids_hbm.at[pl.ds(base, CHUNK)], ids_v[p], isem[p])
        pltpu.async_copy(seg_hbm.at[pl.ds(base, CHUNK)], segs_v[p], ssem[p])

    def wait_ids(n, p):
        base = chunk_base(n)
        pltpu.make_async_copy(ids_hbm.at[pl.ds(base, CHUNK)], ids_v[p],
                              isem[p]).wait()
        pltpu.make_async_copy(seg_hbm.at[pl.ds(base, CHUNK)], segs_v[p],
                              ssem[p]).wait()

    def issue_gathers(n, p):
        base = chunk_base(n)
        for k in range(CHUNK // LANES):
            sl = pl.ds(LANES * k, LANES)
            svec = lax.rem(jnp.full((LANES,), base + LANES * k, jnp.int32)
                           + lax.iota(jnp.int32, LANES), S)
            psidx_v[p][sl] = segs_v[p][sl] * S + svec
        pltpu.async_copy(w_hbm.at[ids_v[p]], rows_v[p], gsem[p])
        pltpu.async_copy(ps_shared.at[psidx_v[p]], psrows_v[p], pssem[p])

    def wait_gathers(p):
        pltpu.make_async_copy(w_hbm.at[ids_v[p]], rows_v[p], gsem[p]).wait()
        pltpu.make_async_copy(ps_shared.at[psidx_v[p]], psrows_v[p],
                              pssem[p]).wait()

    def add_and_writeout(n, p):
        rv, pv = rows_v[p], psrows_v[p]
        base = chunk_base(n)

        @pl.loop(0, CHUNK)
        def _row(r):
            for j in range(VPR):
                sl = pl.ds(16 * j, 16)
                rv[r, sl] = rv[r, sl] + pv[r, sl]

        pltpu.async_copy(rv, out_hbm.at[pl.ds(base, CHUNK)], wsem[p])

    def wait_writeout(n, p):
        pltpu.make_async_copy(rows_v[p],
                              out_hbm.at[pl.ds(chunk_base(n), CHUNK)],
                              wsem[p]).wait()

    for i in range(DEPTH):
        fetch_ids(i, i)

    @pl.when(sid == 0)
    def _build():
        pltpu.sync_copy(p_hbm.at[pl.ds(0, S)], pbuf_v)
        pltpu.sync_copy(sg_hbm, sg_v)
        seg0 = [sg_v[0, pl.ds(16 * j, 16)] for j in range(VPR)]
        dseg = [sg_v[1, pl.ds(16 * j, 16)] - sg_v[0, pl.ds(16 * j, 16)]
                for j in range(VPR)]

        @pl.loop(0, S)
        def _add0(s):
            for j in range(VPR):
                sl = pl.ds(16 * j, 16)
                pbuf_v[s, sl] = pbuf_v[s, sl] + seg0[j]

        pltpu.sync_copy(pbuf_v, ps_shared.at[pl.ds(0, S)])

        @pl.loop(0, S)
        def _add1(s):
            for j in range(VPR):
                sl = pl.ds(16 * j, 16)
                pbuf_v[s, sl] = pbuf_v[s, sl] + dseg[j]

        pltpu.sync_copy(pbuf_v, ps_shared.at[pl.ds(S, S)])

    plsc.subcore_barrier()

    wait_ids(0, 0)
    issue_gathers(0, 0)
    wait_ids(1, 1)
    issue_gathers(1, 1)
    wait_gathers(0)
    fetch_ids(DEPTH, 0)
    add_and_writeout(0, 0)
    wait_ids(2, 2)
    issue_gathers(2, 2)
    wait_gathers(1)
    add_and_writeout(1, 1)

    def steady(n, p):
        q = (p + DEPTH - 1) % DEPTH
        r3 = (p + 1) % DEPTH
        wait_writeout(n - DEPTH, p)
        wait_ids(n, p)
        issue_gathers(n, p)
        wait_gathers(q)
        fetch_ids(n + 1, r3)
        add_and_writeout(n - 1, q)

    @pl.loop(1, (CHUNKS_PER_W - 3) // DEPTH + 1)
    def _grp(m):
        for b in range(DEPTH):
            steady(DEPTH * m + b, b)

    steady(CHUNKS_PER_W - 2, (CHUNKS_PER_W - 2) % DEPTH)
    steady(CHUNKS_PER_W - 1, (CHUNKS_PER_W - 1) % DEPTH)

    last = CHUNKS_PER_W - 1
    wait_writeout(last - 2, (last - 2) % DEPTH)
    wait_gathers(last % DEPTH)
    add_and_writeout(last, last % DEPTH)
    wait_ids(CHUNKS_PER_W, CHUNKS_PER_W % DEPTH)
    wait_writeout(last - 1, (last - 1) % DEPTH)
    wait_writeout(last, last % DEPTH)


@jax.jit
def _run(input_ids, segment_ids, word_embeddings, position_embeddings,
         segment_embeddings):
    ids = input_ids.reshape(N)
    segs = segment_ids.reshape(N)
    mesh = plsc.VectorSubcoreMesh(core_axis_name="c", subcore_axis_name="s",
                                  num_cores=NUM_CORES,
                                  num_subcores=NUM_SUBCORES)
    out = pl.kernel(
        _body,
        out_type=jax.ShapeDtypeStruct((N, D), jnp.float32),
        mesh=mesh,
        scratch_types=[
            pltpu.VMEM_SHARED((NUM_SEG * S, D), jnp.float32),
            pltpu.VMEM((S, D), jnp.float32),
            pltpu.VMEM((NUM_SEG, D), jnp.float32),
        ]
        + [pltpu.VMEM((CHUNK,), jnp.int32)] * 9
        + [pltpu.VMEM((CHUNK, D), jnp.float32)] * 6
        + [pltpu.SemaphoreType.DMA] * 15,
    )(ids, segs, word_embeddings, position_embeddings, segment_embeddings)
    return out.reshape(B, S, D)


def kernel(input_ids, segment_ids, word_embeddings, position_embeddings,
           segment_embeddings):
    return _run(input_ids, segment_ids, word_embeddings,
                position_embeddings, segment_embeddings)


# speedup vs baseline: 3.0914x; 1.0030x over previous
import jax
import jax.numpy as jnp
from jax import lax
from jax.experimental import pallas as pl
from jax.experimental.pallas import tpu as pltpu
from jax.experimental.pallas import tpu_sc as plsc

D = 128
B = 4096
S = 200
NUM_SEG = 2
N = B * S

NUM_CORES = 2
NUM_SUBCORES = 16
NUM_WORKERS = NUM_CORES * NUM_SUBCORES
PER_W = N // NUM_WORKERS
CHUNK = 128
CHUNKS_PER_W = PER_W // CHUNK
LANES = 16
VPR = D // LANES
DEPTH = 3


def _body(ids_hbm, seg_hbm, w_hbm, p_hbm, sg_hbm, out_hbm,
          ps_shared, pbuf_v, sg_v,
          ids0, ids1, ids2, segs0, segs1, segs2, psidx0, psidx1, psidx2,
          rows0, rows1, rows2, psrows0, psrows1, psrows2,
          gsem0, gsem1, gsem2, pssem0, pssem1, pssem2,
          isem0, isem1, isem2, ssem0, ssem1, ssem2,
          wsem0, wsem1, wsem2):
    ids_v = (ids0, ids1, ids2)
    segs_v = (segs0, segs1, segs2)
    psidx_v = (psidx0, psidx1, psidx2)
    rows_v = (rows0, rows1, rows2)
    psrows_v = (psrows0, psrows1, psrows2)
    gsem = (gsem0, gsem1, gsem2)
    pssem = (pssem0, pssem1, pssem2)
    isem = (isem0, isem1, isem2)
    ssem = (ssem0, ssem1, ssem2)
    wsem = (wsem0, wsem1, wsem2)

    cid = lax.axis_index("c")
    sid = lax.axis_index("s")
    wid = sid * NUM_CORES + cid
    wstart = wid * PER_W

    def chunk_base(n):
        return lax.rem(wstart + n * CHUNK, N)

    def fetch_ids(n, p):
        base = chunk_base(n)
        pltpu.async_copy(ids_hbm.at[pl.ds(base, CHUNK)], ids_v[p], isem[p])
        pltpu.async_copy(seg_hbm.at[pl.ds(base, CHUNK)], segs_v[p], ssem[p])

    def wait_ids(n, p):
        base = chunk_base(n)
        pltpu.make_async_copy(ids_hbm.at[pl.ds(base, CHUNK)], ids_v[p],
                              isem[p]).wait()
        pltpu.make_async_copy(seg_hbm.at[pl.ds(base, CHUNK)], segs_v[p],
                              ssem[p]).wait()

    def issue_gathers(n, p):
        base = chunk_base(n)
        pltpu.async_copy(w_hbm.at[ids_v[p]], rows_v[p], gsem[p])
        for k in range(CHUNK // LANES):
            sl = pl.ds(LANES * k, LANES)
            svec = lax.rem(jnp.full((LANES,), base + LANES * k, jnp.int32)
                           + lax.iota(jnp.int32, LANES), S)
            psidx_v[p][sl] = segs_v[p][sl] * S + svec
        pltpu.async_copy(ps_shared.at[psidx_v[p]], psrows_v[p], pssem[p])

    def wait_gathers(p):
        pltpu.make_async_copy(w_hbm.at[ids_v[p]], rows_v[p], gsem[p]).wait()
        pltpu.make_async_copy(ps_shared.at[psidx_v[p]], psrows_v[p],
                              pssem[p]).wait()

    def add_and_writeout(n, p):
        rv, pv = rows_v[p], psrows_v[p]
        base = chunk_base(n)

        @pl.loop(0, CHUNK)
        def _row(r):
            for j in range(VPR):
                sl = pl.ds(16 * j, 16)
                rv[r, sl] = rv[r, sl] + pv[r, sl]

        pltpu.async_copy(rv, out_hbm.at[pl.ds(base, CHUNK)], wsem[p])

    def wait_writeout(n, p):
        pltpu.make_async_copy(rows_v[p],
                              out_hbm.at[pl.ds(chunk_base(n), CHUNK)],
                              wsem[p]).wait()

    for i in range(DEPTH):
        fetch_ids(i, i)

    @pl.when(sid == 0)
    def _build():
        pltpu.sync_copy(p_hbm.at[pl.ds(0, S)], pbuf_v)
        pltpu.sync_copy(sg_hbm, sg_v)
        seg0 = [sg_v[0, pl.ds(16 * j, 16)] for j in range(VPR)]
        dseg = [sg_v[1, pl.ds(16 * j, 16)] - sg_v[0, pl.ds(16 * j, 16)]
                for j in range(VPR)]

        @pl.loop(0, S)
        def _add0(s):
            for j in range(VPR):
                sl = pl.ds(16 * j, 16)
                pbuf_v[s, sl] = pbuf_v[s, sl] + seg0[j]

        pltpu.sync_copy(pbuf_v, ps_shared.at[pl.ds(0, S)])

        @pl.loop(0, S)
        def _add1(s):
            for j in range(VPR):
                sl = pl.ds(16 * j, 16)
                pbuf_v[s, sl] = pbuf_v[s, sl] + dseg[j]

        pltpu.sync_copy(pbuf_v, ps_shared.at[pl.ds(S, S)])

    plsc.subcore_barrier()

    wait_ids(0, 0)
    issue_gathers(0, 0)
    wait_ids(1, 1)
    issue_gathers(1, 1)
    wait_gathers(0)
    fetch_ids(DEPTH, 0)
    add_and_writeout(0, 0)
    wait_ids(2, 2)
    issue_gathers(2, 2)
    wait_gathers(1)
    add_and_writeout(1, 1)

    def steady(n, p):
        q = (p + DEPTH - 1) % DEPTH
        r3 = (p + 1) % DEPTH
        fetch_ids(n + 1, r3)
        wait_writeout(n - DEPTH, p)
        wait_ids(n, p)
        issue_gathers(n, p)
        wait_gathers(q)
        add_and_writeout(n - 1, q)

    @pl.loop(1, (CHUNKS_PER_W - 3) // DEPTH + 1)
    def _grp(m):
        for b in range(DEPTH):
            steady(DEPTH * m + b, b)

    steady(CHUNKS_PER_W - 2, (CHUNKS_PER_W - 2) % DEPTH)
    steady(CHUNKS_PER_W - 1, (CHUNKS_PER_W - 1) % DEPTH)

    last = CHUNKS_PER_W - 1
    wait_writeout(last - 2, (last - 2) % DEPTH)
    wait_gathers(last % DEPTH)
    add_and_writeout(last, last % DEPTH)
    wait_ids(CHUNKS_PER_W, CHUNKS_PER_W % DEPTH)
    wait_writeout(last - 1, (last - 1) % DEPTH)
    wait_writeout(last, last % DEPTH)


@jax.jit
def _run(input_ids, segment_ids, word_embeddings, position_embeddings,
         segment_embeddings):
    ids = input_ids.reshape(N)
    segs = segment_ids.reshape(N)
    mesh = plsc.VectorSubcoreMesh(core_axis_name="c", subcore_axis_name="s",
                                  num_cores=NUM_CORES,
                                  num_subcores=NUM_SUBCORES)
    out = pl.kernel(
        _body,
        out_type=jax.ShapeDtypeStruct((N, D), jnp.float32),
        mesh=mesh,
        scratch_types=[
            pltpu.VMEM_SHARED((NUM_SEG * S, D), jnp.float32),
            pltpu.VMEM((S, D), jnp.float32),
            pltpu.VMEM((NUM_SEG, D), jnp.float32),
        ]
        + [pltpu.VMEM((CHUNK,), jnp.int32)] * 9
        + [pltpu.VMEM((CHUNK, D), jnp.float32)] * 6
        + [pltpu.SemaphoreType.DMA] * 15,
    )(ids, segs, word_embeddings, position_embeddings, segment_embeddings)
    return out.reshape(B, S, D)


def kernel(input_ids, segment_ids, word_embeddings, position_embeddings,
           segment_embeddings):
    return _run(input_ids, segment_ids, word_embeddings,
                position_embeddings, segment_embeddings)
